# baseline (speedup 1.0000x reference)
import jax
import jax.numpy as jnp
import numpy as np
from jax.experimental import pallas as pl
from jax.experimental.pallas import tpu as pltpu

_B, _C, _N = 4, 64, 1024
_U = 10


def _build_selection_matrix() -> np.ndarray:
    with jax.default_device(jax.local_devices(backend="cpu")[0]):
        skey = jax.random.key(42)
        idx = np.asarray(jax.random.randint(skey, (_C, _U), 0, _C))
    e = (idx.T.reshape(_U * _C, 1) == np.arange(_C)[None, :])
    return np.ascontiguousarray(e.astype(np.float32))


_E_SEL = _build_selection_matrix()


def _block_kernel(x_ref, w1t_ref, b1_ref, w2t_ref, b2_ref, gamma_ref, e_ref,
                  out_ref):
    rb = range(x_ref.shape[0])
    xs = [x_ref[b] for b in rb]

    qns = [jnp.sqrt(jnp.sum(xb * xb, axis=0, keepdims=True)) for xb in xs]
    qinvs = [jnp.where(qn > 0.0, 1.0 / qn, 0.0) for qn in qns]
    qhats = [xb * qi for xb, qi in zip(xs, qinvs)]
    sims = [jax.lax.dot_general(qh, qh, (((0,), (0,)), ((), ())),
                                preferred_element_type=jnp.float32)
            for qh in qhats]
    adjs = [(s > 0.5).astype(jnp.float32) for s in sims]
    rinvs = [1.0 / (jnp.sum(a, axis=0, keepdims=True) + 1.0) for a in adjs]

    qrs = [1.0 / jnp.sum(xb, axis=0, keepdims=True) for xb in xs]
    qrs = [jnp.where(jnp.isinf(r), 0.0, r) for r in qrs]
    qfs = [xb * r for xb, r in zip(xs, qrs)]

    w1t, w2t = w1t_ref[...].T, w2t_ref[...].T
    b1c, b2c = b1_ref[...], b2_ref[...]
    h1s = [jnp.dot(w1t, qf, preferred_element_type=jnp.float32) for qf in qfs]
    p1s = [jnp.dot(h1, a, preferred_element_type=jnp.float32) + h1
           for h1, a in zip(h1s, adjs)]
    hs = [jax.nn.relu(p1 * ri + b1c) for p1, ri in zip(p1s, rinvs)]
    h2s = [jnp.dot(w2t, h, preferred_element_type=jnp.float32) for h in hs]
    p2s = [jnp.dot(h2, a, preferred_element_type=jnp.float32) + h2
           for h2, a in zip(h2s, adjs)]
    qrys = [p2 * ri + b2c for p2, ri in zip(p2s, rinvs)]

    ec = e_ref[...]
    gs = [jnp.dot(ec, q, preferred_element_type=jnp.float32) for q in qrys]
    qks = [jnp.sum(g.reshape(_U, _C, _N) * q[None, :, :], axis=1)
           for g, q in zip(gs, qrys)]
    ms = [jnp.max(qk, axis=0, keepdims=True)
          - jnp.sum(qk, axis=0, keepdims=True) * (1.0 / 64.0) for qk in qks]
    m_all = jnp.concatenate(ms, axis=0)

    iota_n = jax.lax.broadcasted_iota(jnp.int32, (len(xs), _N), 1)
    onehot_rows = []
    m_work = m_all
    for _ in range(_U):
        mv = jnp.max(m_work, axis=1, keepdims=True)
        cand = jnp.where(m_work == mv, iota_n, _N)
        sel = jnp.min(cand, axis=1, keepdims=True)
        row = (iota_n == sel)
        onehot_rows.append(row.astype(jnp.float32))
        m_work = jnp.where(row, -jnp.inf, m_work)
    os_ = [jnp.concatenate([r[b:b + 1] for r in onehot_rows], axis=0)
           for b in rb]

    qreds = [jax.lax.dot_general(q, o, (((1,), (1,)), ((), ())),
                                 preferred_element_type=jnp.float32)
             for q, o in zip(qrys, os_)]
    scale = 1.0 / np.sqrt(float(_N))
    scs = [jax.lax.dot_general(qr, q, (((0,), (0,)), ((), ())),
                               preferred_element_type=jnp.float32) * scale
           for qr, q in zip(qreds, qrys)]
    exs = [jnp.exp(s - jnp.max(s, axis=1, keepdims=True)) for s in scs]
    attns = [e / jnp.sum(e, axis=1, keepdims=True) for e in exs]
    upds = [jax.lax.dot_general(q, a, (((1,), (1,)), ((), ())),
                                preferred_element_type=jnp.float32)
            for q, a in zip(qrys, attns)]

    ctxs = list(qrys)
    shift = 1
    while shift < _N:
        z = jnp.zeros((_C, shift), jnp.float32)
        ctxs = [c + jnp.concatenate([z, c[:, :-shift]], axis=1) for c in ctxs]
        shift *= 2
    masks = [jnp.sum(o, axis=0, keepdims=True) for o in os_]
    scats = [jnp.dot(u, o, preferred_element_type=jnp.float32)
             for u, o in zip(upds, os_)]
    gm = gamma_ref[0, 0]
    for b in rb:
        ctx = jnp.where(masks[b] > 0.0, scats[b], ctxs[b])
        out_ref[b] = gm * ctx + xs[b]


_PB = 2


def kernel(x, W1, b1, W2, b2, gamma):
    out = pl.pallas_call(
        _block_kernel,
        grid=(_B // _PB,),
        in_specs=[
            pl.BlockSpec((_PB, _C, _N), lambda i: (i, 0, 0)),
            pl.BlockSpec((_C, 8), lambda i: (0, 0)),
            pl.BlockSpec((8, 1), lambda i: (0, 0)),
            pl.BlockSpec((8, _C), lambda i: (0, 0)),
            pl.BlockSpec((_C, 1), lambda i: (0, 0)),
            pl.BlockSpec((1, 1), lambda i: (0, 0)),
            pl.BlockSpec((_U * _C, _C), lambda i: (0, 0)),
        ],
        out_specs=pl.BlockSpec((_PB, _C, _N), lambda i: (i, 0, 0)),
        out_shape=jax.ShapeDtypeStruct((_B, _C, _N), jnp.float32),
    )(x, W1, b1.reshape(8, 1), W2, b2.reshape(_C, 1),
      gamma.reshape(1, 1), jnp.asarray(_E_SEL))
    return out


# speedup vs baseline: 1.2101x; 1.2101x over previous
import jax
import jax.numpy as jnp
import numpy as np
from jax.experimental import pallas as pl
from jax.experimental.pallas import tpu as pltpu

_B, _C, _N = 4, 64, 1024
_U = 10


def _build_selection_matrix() -> np.ndarray:
    with jax.default_device(jax.local_devices(backend="cpu")[0]):
        skey = jax.random.key(42)
        idx = np.asarray(jax.random.randint(skey, (_C, _U), 0, _C))
    e = (idx.T.reshape(_U * _C, 1) == np.arange(_C)[None, :])
    return np.ascontiguousarray(e.astype(np.float32))


_E_SEL = _build_selection_matrix()


def _block_kernel(x_ref, w1t_ref, b1_ref, w2t_ref, b2_ref, gamma_ref, e_ref,
                  out_ref):
    rb = range(x_ref.shape[0])
    xs = [x_ref[b] for b in rb]

    qns = [jnp.sqrt(jnp.sum(xb * xb, axis=0, keepdims=True)) for xb in xs]
    qinvs = [jnp.where(qn > 0.0, 1.0 / qn, 0.0) for qn in qns]
    qhats = [xb * qi for xb, qi in zip(xs, qinvs)]
    sims = [jax.lax.dot_general(qh, qh, (((0,), (0,)), ((), ())),
                                preferred_element_type=jnp.float32)
            for qh in qhats]
    adjs = [(s > 0.5).astype(jnp.float32) for s in sims]
    rinvs = [1.0 / (jnp.sum(a, axis=0, keepdims=True) + 1.0) for a in adjs]

    qrs = [1.0 / jnp.sum(xb, axis=0, keepdims=True) for xb in xs]
    qrs = [jnp.where(jnp.isinf(r), 0.0, r) for r in qrs]
    qfs = [xb * r for xb, r in zip(xs, qrs)]

    w1t, w2t = w1t_ref[...].T, w2t_ref[...].T
    b1c, b2c = b1_ref[...], b2_ref[...]
    h1s = [jnp.dot(w1t, qf, preferred_element_type=jnp.float32) for qf in qfs]
    p1s = [jnp.dot(h1, a, preferred_element_type=jnp.float32) + h1
           for h1, a in zip(h1s, adjs)]
    hs = [jax.nn.relu(p1 * ri + b1c) for p1, ri in zip(p1s, rinvs)]
    h2s = [jnp.dot(w2t, h, preferred_element_type=jnp.float32) for h in hs]
    p2s = [jnp.dot(h2, a, preferred_element_type=jnp.float32) + h2
           for h2, a in zip(h2s, adjs)]
    qrys = [p2 * ri + b2c for p2, ri in zip(p2s, rinvs)]

    ec = e_ref[...]
    gs = [jnp.dot(ec, q, preferred_element_type=jnp.float32) for q in qrys]
    qks = [jnp.sum(g.reshape(_U, _C, _N) * q[None, :, :], axis=1)
           for g, q in zip(gs, qrys)]
    ms = [jnp.max(qk, axis=0, keepdims=True)
          - jnp.sum(qk, axis=0, keepdims=True) * (1.0 / 64.0) for qk in qks]
    m_all = jnp.concatenate(ms, axis=0)

    iota_n = jax.lax.broadcasted_iota(jnp.int32, (len(xs), _N), 1)
    onehot_rows = []
    m_work = m_all
    for _ in range(_U):
        mv = jnp.max(m_work, axis=1, keepdims=True)
        cand = jnp.where(m_work == mv, iota_n, _N)
        sel = jnp.min(cand, axis=1, keepdims=True)
        row = (iota_n == sel)
        onehot_rows.append(row.astype(jnp.float32))
        m_work = jnp.where(row, -jnp.inf, m_work)
    os_ = [jnp.concatenate([r[b:b + 1] for r in onehot_rows], axis=0)
           for b in rb]

    qreds = [jax.lax.dot_general(q, o, (((1,), (1,)), ((), ())),
                                 preferred_element_type=jnp.float32)
             for q, o in zip(qrys, os_)]
    scale = 1.0 / np.sqrt(float(_N))
    scs = [jax.lax.dot_general(qr, q, (((0,), (0,)), ((), ())),
                               preferred_element_type=jnp.float32) * scale
           for qr, q in zip(qreds, qrys)]
    exs = [jnp.exp(s - jnp.max(s, axis=1, keepdims=True)) for s in scs]
    attns = [e / jnp.sum(e, axis=1, keepdims=True) for e in exs]
    upds = [jax.lax.dot_general(q, a, (((1,), (1,)), ((), ())),
                                preferred_element_type=jnp.float32)
            for q, a in zip(qrys, attns)]

    ctxs = list(qrys)
    shift = 1
    while shift < _N:
        z = jnp.zeros((_C, shift), jnp.float32)
        ctxs = [c + jnp.concatenate([z, c[:, :-shift]], axis=1) for c in ctxs]
        shift *= 2
    masks = [jnp.sum(o, axis=0, keepdims=True) for o in os_]
    scats = [jnp.dot(u, o, preferred_element_type=jnp.float32)
             for u, o in zip(upds, os_)]
    gm = gamma_ref[0, 0]
    for b in rb:
        ctx = jnp.where(masks[b] > 0.0, scats[b], ctxs[b])
        out_ref[b] = gm * ctx + xs[b]


def kernel(x, W1, b1, W2, b2, gamma):
    out = pl.pallas_call(
        _block_kernel,
        out_shape=jax.ShapeDtypeStruct((_B, _C, _N), jnp.float32),
    )(x, W1, b1.reshape(8, 1), W2, b2.reshape(_C, 1),
      gamma.reshape(1, 1), jnp.asarray(_E_SEL))
    return out
